# baseline (speedup 1.0000x reference)
import functools

import jax
import jax.numpy as jnp
from jax import lax
from jax.experimental import pallas as pl
from jax.experimental.pallas import tpu as pltpu
from jax.experimental.pallas import tpu_sc as plsc

SEG = 8
CAP = 1536
NROW = SEG * CAP
KNN = 30
KPAD = 32
RB = 256


def _b16(x):
    return x.astype(jnp.bfloat16)


def _mm(a, b):
    return jnp.dot(_b16(a), _b16(b), preferred_element_type=jnp.float32)


def _knn_body(slen_ref, xr_ref, xw_ref, m_ref, idx_ref, vals_ref):
    b = pl.program_id(0)
    r = pl.program_id(1)
    gbase = b * CAP
    slen = slen_ref[b]

    @pl.when(r * RB < slen)
    def _active():
        xr = xr_ref[0]
        xw = xw_ref[0]
        dot = lax.dot_general(_b16(xr), _b16(xw), (((1,), (1,)), ((), ())),
                              preferred_element_type=jnp.float32)
        rr = jnp.sum(xr * xr, axis=1, keepdims=True)
        ww = jnp.sum(xw * xw, axis=1)[None, :]
        vals_ref[...] = rr + ww - 2.0 * dot + m_ref[0]

        colf = lax.broadcasted_iota(jnp.int32, (RB, CAP), 1).astype(jnp.float32)

        def step(t, carry):
            v = vals_ref[...]
            m = jnp.min(v, axis=1, keepdims=True)
            jf = jnp.min(jnp.where(v == m, colf, jnp.float32(CAP)), axis=1)
            onehot = colf == jf[:, None]
            vals_ref[...] = jnp.where(onehot, jnp.float32(jnp.inf), v)
            idx_ref[0, pl.ds(t, 1), :] = (jf.astype(jnp.int32) + gbase)[None, :]
            return carry

        lax.fori_loop(0, KNN, step, 0)
        idx_ref[0, KNN:KPAD, :] = jnp.zeros((KPAD - KNN, RB), jnp.int32) + gbase

    @pl.when(r * RB >= slen)
    def _padding_rows():
        idx_ref[0] = jnp.zeros((KPAD, RB), jnp.int32) + gbase


def _knn(xp, maskcol, seg_len):
    din = xp.shape[-1]
    grid = (xp.shape[0], CAP // RB)
    return pl.pallas_call(
        _knn_body,
        grid=grid,
        in_specs=[
            pl.BlockSpec(memory_space=pltpu.SMEM),
            pl.BlockSpec((1, RB, din), lambda b, r: (b, r, 0)),
            pl.BlockSpec((1, CAP, din), lambda b, r: (b, 0, 0)),
            pl.BlockSpec((1, 1, CAP), lambda b, r: (b, 0, 0)),
        ],
        out_specs=pl.BlockSpec((1, KPAD, RB), lambda b, r: (b, 0, r)),
        out_shape=jax.ShapeDtypeStruct((xp.shape[0], KPAD, CAP), jnp.int32),
        scratch_shapes=[pltpu.VMEM((RB, CAP), jnp.float32)],
    )(seg_len, xp, xp, maskcol)


def _gather_rows(table, idx):
    n_rows = idx.shape[0]
    d = table.shape[1]
    info = plsc.get_sparse_core_info()
    nw = info.num_cores * info.num_subcores
    per_w = n_rows // nw
    ch = 128
    n_ch = per_w // ch
    mesh = plsc.VectorSubcoreMesh(core_axis_name="c", subcore_axis_name="s")

    nbuf = 8
    ngroups = n_ch // nbuf

    @functools.partial(
        pl.kernel, mesh=mesh,
        compiler_params=pltpu.CompilerParams(use_tc_tiling_on_sc=False),
        out_type=jax.ShapeDtypeStruct((n_rows, d), jnp.float32),
        scratch_types=(
            [pltpu.VMEM((per_w,), jnp.int32)]
            + [pltpu.VMEM((ch, d), jnp.float32) for _ in range(nbuf)]
            + [pltpu.SemaphoreType.DMA for _ in range(2 * nbuf)]
        ),
    )
    def k(table_hbm, idx_hbm, out_hbm, *bufs):
        idx_all = bufs[0]
        rows_v = bufs[1:1 + nbuf]
        gsem = bufs[1 + nbuf:1 + 2 * nbuf]
        ssem = bufs[1 + 2 * nbuf:]
        wid = lax.axis_index("s") * info.num_cores + lax.axis_index("c")
        base = wid * per_w
        pltpu.sync_copy(idx_hbm.at[pl.ds(base, per_w)], idx_all)

        def gath(c, u):
            pltpu.async_copy(table_hbm.at[idx_all.at[pl.ds(c * ch, ch)]],
                             rows_v[u], gsem[u])

        def gwait(c, u):
            pltpu.make_async_copy(table_hbm.at[idx_all.at[pl.ds(c * ch, ch)]],
                                  rows_v[u], gsem[u]).wait()

        def scat(c, u):
            pltpu.async_copy(rows_v[u], out_hbm.at[pl.ds(base + c * ch, ch)],
                             ssem[u])

        def swait(c, u):
            pltpu.make_async_copy(rows_v[u],
                                  out_hbm.at[pl.ds(base + c * ch, ch)],
                                  ssem[u]).wait()

        for u in range(nbuf):
            gath(u, u)

        def body(g, carry):
            for u in range(nbuf):
                gwait(g * nbuf + u, u)
                scat(g * nbuf + u, u)

            @pl.when(g < ngroups - 1)
            def _refill():
                for u in range(nbuf):
                    swait(g * nbuf + u, u)
                    gath((g + 1) * nbuf + u, u)

            return carry

        lax.fori_loop(0, ngroups, body, 0)
        for u in range(nbuf):
            swait((ngroups - 1) * nbuf + u, u)

    return k(table, idx)


def _edge_body(xi_ref, g_ref, w0a_ref, w0b_ref, b0_ref, w1_ref, b1_ref,
               o_ref):
    din = xi_ref.shape[-1]
    xi = xi_ref[0]
    g = g_ref[0]
    e = (g - xi[None, :, :]).reshape(KPAD * RB, din)
    h1 = _mm(xi, w0a_ref[...])[None, :, :] + (
        _mm(e, w0b_ref[...]).reshape(KPAD, RB, 64))
    h1 = jnp.maximum(h1 + b0_ref[...], 0.0).reshape(KPAD * RB, 64)
    h2 = jnp.maximum(_mm(h1, w1_ref[...]) + b1_ref[...], 0.0)
    o_ref[0] = jnp.max(h2.reshape(KPAD, RB, 64)[:KNN], axis=0)


def _edge_mlp_max(xp, g4, w0a, w0b, b0, w1, b1):
    din = xp.shape[-1]
    grid = (xp.shape[0], CAP // RB)
    return pl.pallas_call(
        _edge_body,
        grid=grid,
        in_specs=[
            pl.BlockSpec((1, RB, din), lambda b, r: (b, r, 0)),
            pl.BlockSpec((1, KPAD, RB, din), lambda b, r: (b, 0, r, 0)),
            pl.BlockSpec((din, 64), lambda b, r: (0, 0)),
            pl.BlockSpec((din, 64), lambda b, r: (0, 0)),
            pl.BlockSpec((1, 64), lambda b, r: (0, 0)),
            pl.BlockSpec((64, 64), lambda b, r: (0, 0)),
            pl.BlockSpec((1, 64), lambda b, r: (0, 0)),
        ],
        out_specs=pl.BlockSpec((1, RB, 64), lambda b, r: (b, r, 0)),
        out_shape=jax.ShapeDtypeStruct((xp.shape[0], CAP, 64), jnp.float32),
    )(xp, g4, w0a, w0b, b0, w1, b1)


def _head_body(x1_ref, x2_ref, x3_ref, w0_ref, b0_ref, w1_ref, b1_ref,
               w2_ref, b2_ref, wf_ref, bf_ref, o_ref):
    w0 = w0_ref[...]
    h = (_mm(x1_ref[...], w0[0:64]) + _mm(x2_ref[...], w0[64:128])
         + _mm(x3_ref[...], w0[128:192]) + b0_ref[...])
    h = jnp.maximum(h, 0.0)
    h = jnp.maximum(_mm(h, w1_ref[...]) + b1_ref[...], 0.0)
    h = jnp.maximum(_mm(h, w2_ref[...]) + b2_ref[...], 0.0)
    o_ref[...] = _mm(h, wf_ref[...]) + bf_ref[...]


def _head(x1f, x2f, x3f, w0, b0, w1, b1, w2, b2, wf, bf):
    grid = (NROW // RB,)
    c = lambda i: (0, 0)
    return pl.pallas_call(
        _head_body,
        grid=grid,
        in_specs=[
            pl.BlockSpec((RB, 64), lambda i: (i, 0)),
            pl.BlockSpec((RB, 64), lambda i: (i, 0)),
            pl.BlockSpec((RB, 64), lambda i: (i, 0)),
            pl.BlockSpec((192, 1024), c),
            pl.BlockSpec((1, 1024), c),
            pl.BlockSpec((1024, 256), c),
            pl.BlockSpec((1, 256), c),
            pl.BlockSpec((256, 128), c),
            pl.BlockSpec((1, 128), c),
            pl.BlockSpec((128, 3), c),
            pl.BlockSpec((1, 3), c),
        ],
        out_specs=pl.BlockSpec((RB, 3), lambda i: (i, 0)),
        out_shape=jax.ShapeDtypeStruct((NROW, 3), jnp.float32),
    )(x1f, x2f, x3f, w0, b0, w1, b1, w2, b2, wf, bf)


def _edge_conv(xp, maskcol, seg_len, w0, b0, w1, b1):
    din_real = w0.shape[0] // 2
    din = xp.shape[-1]
    w0a, w0b = w0[:din_real], w0[din_real:]
    if din_real < din:
        padr = din - din_real
        w0a = jnp.pad(w0a, ((0, padr), (0, 0)))
        w0b = jnp.pad(w0b, ((0, padr), (0, 0)))
    nseg = xp.shape[0]
    idx = _knn(xp, maskcol, seg_len)
    g = _gather_rows(xp.reshape(nseg * CAP, din),
                     idx.reshape(nseg * KPAD * CAP))
    g4 = g.reshape(nseg, KPAD, CAP, din)
    return _edge_mlp_max(xp, g4, w0a, w0b, b0.reshape(1, 64), w1,
                         b1.reshape(1, 64))


def kernel(pos, batch, c1_w0, c1_b0, c1_w1, c1_b1, c2_w0, c2_b0, c2_w1,
           c2_b1, c3_w0, c3_b0, c3_w1, c3_b1, mlp_w0, mlp_b0, mlp_w1,
           mlp_b1, mlp_w2, mlp_b2, fin_w, fin_b):
    n = pos.shape[0]
    batch = batch.astype(jnp.int32)
    seg_ids = jnp.arange(SEG, dtype=jnp.int32)
    seg_start = jnp.searchsorted(batch, seg_ids, side="left").astype(jnp.int32)
    seg_len = (jnp.searchsorted(batch, seg_ids, side="right").astype(jnp.int32)
               - seg_start)
    lidx = jnp.arange(CAP, dtype=jnp.int32)[None, :]
    valid = lidx < seg_len[:, None]
    g2l = jnp.where(valid, seg_start[:, None] + lidx, 0)
    maskcol = jnp.where(valid, 0.0, jnp.inf).astype(jnp.float32)
    maskcol = maskcol.reshape(SEG, 1, CAP)

    posp = jnp.pad(pos, ((0, CAP), (0, 5)))
    xp = jnp.stack([lax.dynamic_slice(posp, (seg_start[b], 0), (CAP, 8))
                    for b in range(SEG)])
    xp = jnp.where(valid[..., None], xp, 0.0)

    H = SEG // 2
    xs = [xp[:H], xp[H:]]
    mcs = [maskcol[:H], maskcol[H:]]
    sls = [seg_len[:H], seg_len[H:]]
    x123 = []
    for h in range(2):
        x1 = _edge_conv(xs[h], mcs[h], sls[h], c1_w0, c1_b0, c1_w1, c1_b1)
        x2 = _edge_conv(x1, mcs[h], sls[h], c2_w0, c2_b0, c2_w1, c2_b1)
        x3 = _edge_conv(x2, mcs[h], sls[h], c3_w0, c3_b0, c3_w1, c3_b1)
        x123.append((x1, x2, x3))
    x1 = jnp.concatenate([x123[0][0], x123[1][0]])
    x2 = jnp.concatenate([x123[0][1], x123[1][1]])
    x3 = jnp.concatenate([x123[0][2], x123[1][2]])

    out = _head(x1.reshape(NROW, 64), x2.reshape(NROW, 64),
                x3.reshape(NROW, 64), mlp_w0, mlp_b0.reshape(1, 1024),
                mlp_w1, mlp_b1.reshape(1, 256), mlp_w2,
                mlp_b2.reshape(1, 128), fin_w, fin_b.reshape(1, 3))

    rows = jnp.arange(n, dtype=jnp.int32)
    out_idx = batch * CAP + (rows - seg_start[batch])
    return jnp.take(out, out_idx, axis=0)


# speedup vs baseline: 1.0714x; 1.0714x over previous
import functools

import jax
import jax.numpy as jnp
from jax import lax
from jax.experimental import pallas as pl
from jax.experimental.pallas import tpu as pltpu
from jax.experimental.pallas import tpu_sc as plsc

SEG = 8
CAP = 1536
NROW = SEG * CAP
KNN = 30
KPAD = 32
RB = 256


def _b16(x):
    return x.astype(jnp.bfloat16)


def _mm(a, b):
    return jnp.dot(_b16(a), _b16(b), preferred_element_type=jnp.float32)


def _knn_body(slen_ref, xr_ref, xw_ref, m_ref, idx_ref, vals_ref):
    b = pl.program_id(0)
    r = pl.program_id(1)
    gbase = b * CAP
    slen = slen_ref[b]

    @pl.when(r * RB < slen)
    def _active():
        xr = xr_ref[0]
        xw = xw_ref[0]
        dot = lax.dot_general(_b16(xr), _b16(xw), (((1,), (1,)), ((), ())),
                              preferred_element_type=jnp.float32)
        rr = jnp.sum(xr * xr, axis=1, keepdims=True)
        ww = jnp.sum(xw * xw, axis=1)[None, :]
        vals_ref[...] = rr + ww - 2.0 * dot + m_ref[0]

        colf = lax.broadcasted_iota(jnp.int32, (RB, CAP), 1).astype(jnp.float32)

        def step(t, carry):
            v = vals_ref[...]
            m = jnp.min(v, axis=1, keepdims=True)
            jf = jnp.min(jnp.where(v == m, colf, jnp.float32(CAP)), axis=1)
            onehot = colf == jf[:, None]
            vals_ref[...] = jnp.where(onehot, jnp.float32(jnp.inf), v)
            idx_ref[0, pl.ds(t, 1), :] = (jf.astype(jnp.int32) + gbase)[None, :]
            return carry

        lax.fori_loop(0, KNN, step, 0, unroll=2)
        idx_ref[0, KNN:KPAD, :] = jnp.zeros((KPAD - KNN, RB), jnp.int32) + gbase

    @pl.when(r * RB >= slen)
    def _padding_rows():
        idx_ref[0] = jnp.zeros((KPAD, RB), jnp.int32) + gbase


def _knn(xp, maskcol, seg_len):
    din = xp.shape[-1]
    grid = (xp.shape[0], CAP // RB)
    return pl.pallas_call(
        _knn_body,
        grid=grid,
        in_specs=[
            pl.BlockSpec(memory_space=pltpu.SMEM),
            pl.BlockSpec((1, RB, din), lambda b, r: (b, r, 0)),
            pl.BlockSpec((1, CAP, din), lambda b, r: (b, 0, 0)),
            pl.BlockSpec((1, 1, CAP), lambda b, r: (b, 0, 0)),
        ],
        out_specs=pl.BlockSpec((1, KPAD, RB), lambda b, r: (b, 0, r)),
        out_shape=jax.ShapeDtypeStruct((xp.shape[0], KPAD, CAP), jnp.int32),
        scratch_shapes=[pltpu.VMEM((RB, CAP), jnp.float32)],
    )(seg_len, xp, xp, maskcol)


def _gather_rows(table, idx):
    n_rows = idx.shape[0]
    d = table.shape[1]
    info = plsc.get_sparse_core_info()
    nw = info.num_cores * info.num_subcores
    per_w = n_rows // nw
    ch = 128
    n_ch = per_w // ch
    mesh = plsc.VectorSubcoreMesh(core_axis_name="c", subcore_axis_name="s")

    nbuf = 8
    ngroups = n_ch // nbuf

    @functools.partial(
        pl.kernel, mesh=mesh,
        compiler_params=pltpu.CompilerParams(use_tc_tiling_on_sc=False),
        out_type=jax.ShapeDtypeStruct((n_rows, d), jnp.float32),
        scratch_types=(
            [pltpu.VMEM((per_w,), jnp.int32)]
            + [pltpu.VMEM((ch, d), jnp.float32) for _ in range(nbuf)]
            + [pltpu.SemaphoreType.DMA for _ in range(2 * nbuf)]
        ),
    )
    def k(table_hbm, idx_hbm, out_hbm, *bufs):
        idx_all = bufs[0]
        rows_v = bufs[1:1 + nbuf]
        gsem = bufs[1 + nbuf:1 + 2 * nbuf]
        ssem = bufs[1 + 2 * nbuf:]
        wid = lax.axis_index("s") * info.num_cores + lax.axis_index("c")
        base = wid * per_w
        pltpu.sync_copy(idx_hbm.at[pl.ds(base, per_w)], idx_all)

        def gath(c, u):
            pltpu.async_copy(table_hbm.at[idx_all.at[pl.ds(c * ch, ch)]],
                             rows_v[u], gsem[u])

        def gwait(c, u):
            pltpu.make_async_copy(table_hbm.at[idx_all.at[pl.ds(c * ch, ch)]],
                                  rows_v[u], gsem[u]).wait()

        def scat(c, u):
            pltpu.async_copy(rows_v[u], out_hbm.at[pl.ds(base + c * ch, ch)],
                             ssem[u])

        def swait(c, u):
            pltpu.make_async_copy(rows_v[u],
                                  out_hbm.at[pl.ds(base + c * ch, ch)],
                                  ssem[u]).wait()

        for u in range(nbuf):
            gath(u, u)

        def body(g, carry):
            for u in range(nbuf):
                gwait(g * nbuf + u, u)
                scat(g * nbuf + u, u)

            @pl.when(g < ngroups - 1)
            def _refill():
                for u in range(nbuf):
                    swait(g * nbuf + u, u)
                    gath((g + 1) * nbuf + u, u)

            return carry

        lax.fori_loop(0, ngroups, body, 0)
        for u in range(nbuf):
            swait((ngroups - 1) * nbuf + u, u)

    return k(table, idx)


def _edge_body(xi_ref, g_ref, w0a_ref, w0b_ref, b0_ref, w1_ref, b1_ref,
               o_ref):
    din = xi_ref.shape[-1]
    xi = xi_ref[0]
    g = g_ref[0]
    e = (g - xi[None, :, :]).reshape(KPAD * RB, din)
    h1 = _mm(xi, w0a_ref[...])[None, :, :] + (
        _mm(e, w0b_ref[...]).reshape(KPAD, RB, 64))
    h1 = jnp.maximum(h1 + b0_ref[...], 0.0).reshape(KPAD * RB, 64)
    h2 = jnp.maximum(_mm(h1, w1_ref[...]) + b1_ref[...], 0.0)
    o_ref[0] = jnp.max(h2.reshape(KPAD, RB, 64)[:KNN], axis=0)


def _edge_mlp_max(xp, g4, w0a, w0b, b0, w1, b1):
    din = xp.shape[-1]
    grid = (xp.shape[0], CAP // RB)
    return pl.pallas_call(
        _edge_body,
        grid=grid,
        in_specs=[
            pl.BlockSpec((1, RB, din), lambda b, r: (b, r, 0)),
            pl.BlockSpec((1, KPAD, RB, din), lambda b, r: (b, 0, r, 0)),
            pl.BlockSpec((din, 64), lambda b, r: (0, 0)),
            pl.BlockSpec((din, 64), lambda b, r: (0, 0)),
            pl.BlockSpec((1, 64), lambda b, r: (0, 0)),
            pl.BlockSpec((64, 64), lambda b, r: (0, 0)),
            pl.BlockSpec((1, 64), lambda b, r: (0, 0)),
        ],
        out_specs=pl.BlockSpec((1, RB, 64), lambda b, r: (b, r, 0)),
        out_shape=jax.ShapeDtypeStruct((xp.shape[0], CAP, 64), jnp.float32),
    )(xp, g4, w0a, w0b, b0, w1, b1)


def _head_body(x1_ref, x2_ref, x3_ref, w0_ref, b0_ref, w1_ref, b1_ref,
               w2_ref, b2_ref, wf_ref, bf_ref, o_ref):
    w0 = w0_ref[...]
    h = (_mm(x1_ref[...], w0[0:64]) + _mm(x2_ref[...], w0[64:128])
         + _mm(x3_ref[...], w0[128:192]) + b0_ref[...])
    h = jnp.maximum(h, 0.0)
    h = jnp.maximum(_mm(h, w1_ref[...]) + b1_ref[...], 0.0)
    h = jnp.maximum(_mm(h, w2_ref[...]) + b2_ref[...], 0.0)
    o_ref[...] = _mm(h, wf_ref[...]) + bf_ref[...]


def _head(x1f, x2f, x3f, w0, b0, w1, b1, w2, b2, wf, bf):
    grid = (NROW // RB,)
    c = lambda i: (0, 0)
    return pl.pallas_call(
        _head_body,
        grid=grid,
        in_specs=[
            pl.BlockSpec((RB, 64), lambda i: (i, 0)),
            pl.BlockSpec((RB, 64), lambda i: (i, 0)),
            pl.BlockSpec((RB, 64), lambda i: (i, 0)),
            pl.BlockSpec((192, 1024), c),
            pl.BlockSpec((1, 1024), c),
            pl.BlockSpec((1024, 256), c),
            pl.BlockSpec((1, 256), c),
            pl.BlockSpec((256, 128), c),
            pl.BlockSpec((1, 128), c),
            pl.BlockSpec((128, 3), c),
            pl.BlockSpec((1, 3), c),
        ],
        out_specs=pl.BlockSpec((RB, 3), lambda i: (i, 0)),
        out_shape=jax.ShapeDtypeStruct((NROW, 3), jnp.float32),
    )(x1f, x2f, x3f, w0, b0, w1, b1, w2, b2, wf, bf)


def _edge_conv(xp, maskcol, seg_len, w0, b0, w1, b1):
    din_real = w0.shape[0] // 2
    din = xp.shape[-1]
    w0a, w0b = w0[:din_real], w0[din_real:]
    if din_real < din:
        padr = din - din_real
        w0a = jnp.pad(w0a, ((0, padr), (0, 0)))
        w0b = jnp.pad(w0b, ((0, padr), (0, 0)))
    nseg = xp.shape[0]
    idx = _knn(xp, maskcol, seg_len)
    g = _gather_rows(xp.reshape(nseg * CAP, din),
                     idx.reshape(nseg * KPAD * CAP))
    g4 = g.reshape(nseg, KPAD, CAP, din)
    return _edge_mlp_max(xp, g4, w0a, w0b, b0.reshape(1, 64), w1,
                         b1.reshape(1, 64))


def kernel(pos, batch, c1_w0, c1_b0, c1_w1, c1_b1, c2_w0, c2_b0, c2_w1,
           c2_b1, c3_w0, c3_b0, c3_w1, c3_b1, mlp_w0, mlp_b0, mlp_w1,
           mlp_b1, mlp_w2, mlp_b2, fin_w, fin_b):
    n = pos.shape[0]
    batch = batch.astype(jnp.int32)
    seg_ids = jnp.arange(SEG, dtype=jnp.int32)
    seg_start = jnp.searchsorted(batch, seg_ids, side="left").astype(jnp.int32)
    seg_len = (jnp.searchsorted(batch, seg_ids, side="right").astype(jnp.int32)
               - seg_start)
    lidx = jnp.arange(CAP, dtype=jnp.int32)[None, :]
    valid = lidx < seg_len[:, None]
    g2l = jnp.where(valid, seg_start[:, None] + lidx, 0)
    maskcol = jnp.where(valid, 0.0, jnp.inf).astype(jnp.float32)
    maskcol = maskcol.reshape(SEG, 1, CAP)

    posp = jnp.pad(pos, ((0, CAP), (0, 5)))
    xp = jnp.stack([lax.dynamic_slice(posp, (seg_start[b], 0), (CAP, 8))
                    for b in range(SEG)])
    xp = jnp.where(valid[..., None], xp, 0.0)

    x1 = _edge_conv(xp, maskcol, seg_len, c1_w0, c1_b0, c1_w1, c1_b1)
    x2 = _edge_conv(x1, maskcol, seg_len, c2_w0, c2_b0, c2_w1, c2_b1)
    x3 = _edge_conv(x2, maskcol, seg_len, c3_w0, c3_b0, c3_w1, c3_b1)

    out = _head(x1.reshape(NROW, 64), x2.reshape(NROW, 64),
                x3.reshape(NROW, 64), mlp_w0, mlp_b0.reshape(1, 1024),
                mlp_w1, mlp_b1.reshape(1, 256), mlp_w2,
                mlp_b2.reshape(1, 128), fin_w, fin_b.reshape(1, 3))

    rows = jnp.arange(n, dtype=jnp.int32)
    out_idx = batch * CAP + (rows - seg_start[batch])
    return jnp.take(out, out_idx, axis=0)


# speedup vs baseline: 1.1084x; 1.0346x over previous
import functools

import jax
import jax.numpy as jnp
from jax import lax
from jax.experimental import pallas as pl
from jax.experimental.pallas import tpu as pltpu
from jax.experimental.pallas import tpu_sc as plsc

SEG = 8
CAP = 1536
NROW = SEG * CAP
KNN = 30
KPAD = 32
RB = 256


def _b16(x):
    return x.astype(jnp.bfloat16)


def _mm(a, b):
    return jnp.dot(_b16(a), _b16(b), preferred_element_type=jnp.float32)


def _knn_body(slen_ref, xr_ref, xw_ref, m_ref, idx_ref, vals_ref):
    b = pl.program_id(0)
    r = pl.program_id(1)
    gbase = b * CAP
    slen = slen_ref[b]

    @pl.when(r * RB < slen)
    def _active():
        xr = xr_ref[0]
        xw = xw_ref[0]
        dot = lax.dot_general(_b16(xr), _b16(xw), (((1,), (1,)), ((), ())),
                              preferred_element_type=jnp.float32)
        rr = jnp.sum(xr * xr, axis=1, keepdims=True)
        ww = jnp.sum(xw * xw, axis=1)[None, :]
        vals_ref[...] = rr + ww - 2.0 * dot + m_ref[0]

        colf = lax.broadcasted_iota(jnp.int32, (RB, CAP), 1).astype(jnp.float32)

        def step(t, carry):
            v = vals_ref[...]
            m = jnp.min(v, axis=1, keepdims=True)
            jf = jnp.min(jnp.where(v == m, colf, jnp.float32(CAP)), axis=1)
            onehot = colf == jf[:, None]
            vals_ref[...] = jnp.where(onehot, jnp.float32(jnp.inf), v)
            idx_ref[0, pl.ds(t, 1), :] = (jf.astype(jnp.int32) + gbase)[None, :]
            return carry

        lax.fori_loop(0, KNN, step, 0, unroll=5)
        idx_ref[0, KNN:KPAD, :] = jnp.zeros((KPAD - KNN, RB), jnp.int32) + gbase

    @pl.when(r * RB >= slen)
    def _padding_rows():
        idx_ref[0] = jnp.zeros((KPAD, RB), jnp.int32) + gbase


def _knn(xp, maskcol, seg_len):
    din = xp.shape[-1]
    grid = (xp.shape[0], CAP // RB)
    return pl.pallas_call(
        _knn_body,
        grid=grid,
        in_specs=[
            pl.BlockSpec(memory_space=pltpu.SMEM),
            pl.BlockSpec((1, RB, din), lambda b, r: (b, r, 0)),
            pl.BlockSpec((1, CAP, din), lambda b, r: (b, 0, 0)),
            pl.BlockSpec((1, 1, CAP), lambda b, r: (b, 0, 0)),
        ],
        out_specs=pl.BlockSpec((1, KPAD, RB), lambda b, r: (b, 0, r)),
        out_shape=jax.ShapeDtypeStruct((xp.shape[0], KPAD, CAP), jnp.int32),
        scratch_shapes=[pltpu.VMEM((RB, CAP), jnp.float32)],
    )(seg_len, xp, xp, maskcol)


def _gather_rows(table, idx):
    n_rows = idx.shape[0]
    d = table.shape[1]
    info = plsc.get_sparse_core_info()
    nw = info.num_cores * info.num_subcores
    per_w = n_rows // nw
    ch = 128
    n_ch = per_w // ch
    mesh = plsc.VectorSubcoreMesh(core_axis_name="c", subcore_axis_name="s")

    nbuf = 8
    ngroups = n_ch // nbuf

    @functools.partial(
        pl.kernel, mesh=mesh,
        compiler_params=pltpu.CompilerParams(use_tc_tiling_on_sc=False),
        out_type=jax.ShapeDtypeStruct((n_rows, d), jnp.float32),
        scratch_types=(
            [pltpu.VMEM((per_w,), jnp.int32)]
            + [pltpu.VMEM((ch, d), jnp.float32) for _ in range(nbuf)]
            + [pltpu.SemaphoreType.DMA for _ in range(2 * nbuf)]
        ),
    )
    def k(table_hbm, idx_hbm, out_hbm, *bufs):
        idx_all = bufs[0]
        rows_v = bufs[1:1 + nbuf]
        gsem = bufs[1 + nbuf:1 + 2 * nbuf]
        ssem = bufs[1 + 2 * nbuf:]
        wid = lax.axis_index("s") * info.num_cores + lax.axis_index("c")
        base = wid * per_w
        pltpu.sync_copy(idx_hbm.at[pl.ds(base, per_w)], idx_all)

        def gath(c, u):
            pltpu.async_copy(table_hbm.at[idx_all.at[pl.ds(c * ch, ch)]],
                             rows_v[u], gsem[u])

        def gwait(c, u):
            pltpu.make_async_copy(table_hbm.at[idx_all.at[pl.ds(c * ch, ch)]],
                                  rows_v[u], gsem[u]).wait()

        def scat(c, u):
            pltpu.async_copy(rows_v[u], out_hbm.at[pl.ds(base + c * ch, ch)],
                             ssem[u])

        def swait(c, u):
            pltpu.make_async_copy(rows_v[u],
                                  out_hbm.at[pl.ds(base + c * ch, ch)],
                                  ssem[u]).wait()

        for u in range(nbuf):
            gath(u, u)

        def body(g, carry):
            for u in range(nbuf):
                gwait(g * nbuf + u, u)
                scat(g * nbuf + u, u)

            @pl.when(g < ngroups - 1)
            def _refill():
                for u in range(nbuf):
                    swait(g * nbuf + u, u)
                    gath((g + 1) * nbuf + u, u)

            return carry

        lax.fori_loop(0, ngroups, body, 0)
        for u in range(nbuf):
            swait((ngroups - 1) * nbuf + u, u)

    return k(table, idx)


def _edge_body(xi_ref, g_ref, w0a_ref, w0b_ref, b0_ref, w1_ref, b1_ref,
               o_ref):
    din = xi_ref.shape[-1]
    xi = xi_ref[0]
    g = g_ref[0]
    e = (g - xi[None, :, :]).reshape(KPAD * RB, din)
    h1 = _mm(xi, w0a_ref[...])[None, :, :] + (
        _mm(e, w0b_ref[...]).reshape(KPAD, RB, 64))
    h1 = jnp.maximum(h1 + b0_ref[...], 0.0).reshape(KPAD * RB, 64)
    h2 = jnp.maximum(_mm(h1, w1_ref[...]) + b1_ref[...], 0.0)
    o_ref[0] = jnp.max(h2.reshape(KPAD, RB, 64)[:KNN], axis=0)


def _edge_mlp_max(xp, g4, w0a, w0b, b0, w1, b1):
    din = xp.shape[-1]
    grid = (xp.shape[0], CAP // RB)
    return pl.pallas_call(
        _edge_body,
        grid=grid,
        in_specs=[
            pl.BlockSpec((1, RB, din), lambda b, r: (b, r, 0)),
            pl.BlockSpec((1, KPAD, RB, din), lambda b, r: (b, 0, r, 0)),
            pl.BlockSpec((din, 64), lambda b, r: (0, 0)),
            pl.BlockSpec((din, 64), lambda b, r: (0, 0)),
            pl.BlockSpec((1, 64), lambda b, r: (0, 0)),
            pl.BlockSpec((64, 64), lambda b, r: (0, 0)),
            pl.BlockSpec((1, 64), lambda b, r: (0, 0)),
        ],
        out_specs=pl.BlockSpec((1, RB, 64), lambda b, r: (b, r, 0)),
        out_shape=jax.ShapeDtypeStruct((xp.shape[0], CAP, 64), jnp.float32),
    )(xp, g4, w0a, w0b, b0, w1, b1)


def _head_body(x1_ref, x2_ref, x3_ref, w0_ref, b0_ref, w1_ref, b1_ref,
               w2_ref, b2_ref, wf_ref, bf_ref, o_ref):
    w0 = w0_ref[...]
    h = (_mm(x1_ref[...], w0[0:64]) + _mm(x2_ref[...], w0[64:128])
         + _mm(x3_ref[...], w0[128:192]) + b0_ref[...])
    h = jnp.maximum(h, 0.0)
    h = jnp.maximum(_mm(h, w1_ref[...]) + b1_ref[...], 0.0)
    h = jnp.maximum(_mm(h, w2_ref[...]) + b2_ref[...], 0.0)
    o_ref[...] = _mm(h, wf_ref[...]) + bf_ref[...]


def _head(x1f, x2f, x3f, w0, b0, w1, b1, w2, b2, wf, bf):
    grid = (NROW // RB,)
    c = lambda i: (0, 0)
    return pl.pallas_call(
        _head_body,
        grid=grid,
        in_specs=[
            pl.BlockSpec((RB, 64), lambda i: (i, 0)),
            pl.BlockSpec((RB, 64), lambda i: (i, 0)),
            pl.BlockSpec((RB, 64), lambda i: (i, 0)),
            pl.BlockSpec((192, 1024), c),
            pl.BlockSpec((1, 1024), c),
            pl.BlockSpec((1024, 256), c),
            pl.BlockSpec((1, 256), c),
            pl.BlockSpec((256, 128), c),
            pl.BlockSpec((1, 128), c),
            pl.BlockSpec((128, 3), c),
            pl.BlockSpec((1, 3), c),
        ],
        out_specs=pl.BlockSpec((RB, 3), lambda i: (i, 0)),
        out_shape=jax.ShapeDtypeStruct((NROW, 3), jnp.float32),
    )(x1f, x2f, x3f, w0, b0, w1, b1, w2, b2, wf, bf)


def _edge_conv(xp, maskcol, seg_len, w0, b0, w1, b1):
    din_real = w0.shape[0] // 2
    din = xp.shape[-1]
    w0a, w0b = w0[:din_real], w0[din_real:]
    if din_real < din:
        padr = din - din_real
        w0a = jnp.pad(w0a, ((0, padr), (0, 0)))
        w0b = jnp.pad(w0b, ((0, padr), (0, 0)))
    nseg = xp.shape[0]
    idx = _knn(xp, maskcol, seg_len)
    g = _gather_rows(xp.reshape(nseg * CAP, din),
                     idx.reshape(nseg * KPAD * CAP))
    g4 = g.reshape(nseg, KPAD, CAP, din)
    return _edge_mlp_max(xp, g4, w0a, w0b, b0.reshape(1, 64), w1,
                         b1.reshape(1, 64))


def kernel(pos, batch, c1_w0, c1_b0, c1_w1, c1_b1, c2_w0, c2_b0, c2_w1,
           c2_b1, c3_w0, c3_b0, c3_w1, c3_b1, mlp_w0, mlp_b0, mlp_w1,
           mlp_b1, mlp_w2, mlp_b2, fin_w, fin_b):
    n = pos.shape[0]
    batch = batch.astype(jnp.int32)
    seg_ids = jnp.arange(SEG, dtype=jnp.int32)
    seg_start = jnp.searchsorted(batch, seg_ids, side="left").astype(jnp.int32)
    seg_len = (jnp.searchsorted(batch, seg_ids, side="right").astype(jnp.int32)
               - seg_start)
    lidx = jnp.arange(CAP, dtype=jnp.int32)[None, :]
    valid = lidx < seg_len[:, None]
    g2l = jnp.where(valid, seg_start[:, None] + lidx, 0)
    maskcol = jnp.where(valid, 0.0, jnp.inf).astype(jnp.float32)
    maskcol = maskcol.reshape(SEG, 1, CAP)

    posp = jnp.pad(pos, ((0, CAP), (0, 5)))
    xp = jnp.stack([lax.dynamic_slice(posp, (seg_start[b], 0), (CAP, 8))
                    for b in range(SEG)])
    xp = jnp.where(valid[..., None], xp, 0.0)

    x1 = _edge_conv(xp, maskcol, seg_len, c1_w0, c1_b0, c1_w1, c1_b1)
    x2 = _edge_conv(x1, maskcol, seg_len, c2_w0, c2_b0, c2_w1, c2_b1)
    x3 = _edge_conv(x2, maskcol, seg_len, c3_w0, c3_b0, c3_w1, c3_b1)

    out = _head(x1.reshape(NROW, 64), x2.reshape(NROW, 64),
                x3.reshape(NROW, 64), mlp_w0, mlp_b0.reshape(1, 1024),
                mlp_w1, mlp_b1.reshape(1, 256), mlp_w2,
                mlp_b2.reshape(1, 128), fin_w, fin_b.reshape(1, 3))

    rows = jnp.arange(n, dtype=jnp.int32)
    out_idx = batch * CAP + (rows - seg_start[batch])
    return jnp.take(out, out_idx, axis=0)


# speedup vs baseline: 1.1222x; 1.0124x over previous
import functools

import jax
import jax.numpy as jnp
from jax import lax
from jax.experimental import pallas as pl
from jax.experimental.pallas import tpu as pltpu
from jax.experimental.pallas import tpu_sc as plsc

SEG = 8
CAP = 1536
NROW = SEG * CAP
KNN = 30
KPAD = 32
RB = 256


def _b16(x):
    return x.astype(jnp.bfloat16)


def _mm(a, b):
    return jnp.dot(_b16(a), _b16(b), preferred_element_type=jnp.float32)


def _knn_body(slen_ref, xr_ref, xw_ref, m_ref, idx_ref, vals_ref):
    b = pl.program_id(0)
    r = pl.program_id(1)
    gbase = b * CAP
    slen = slen_ref[b]

    @pl.when(r * RB < slen)
    def _active():
        xr = xr_ref[0]
        xw = xw_ref[0]
        dot = lax.dot_general(_b16(xr), _b16(xw), (((1,), (1,)), ((), ())),
                              preferred_element_type=jnp.float32)
        rr = jnp.sum(xr * xr, axis=1, keepdims=True)
        ww = jnp.sum(xw * xw, axis=1)[None, :]
        vals_ref[...] = rr + ww - 2.0 * dot + m_ref[0]

        colf = lax.broadcasted_iota(jnp.int32, (RB, CAP), 1).astype(jnp.float32)

        def step(t, carry):
            v = vals_ref[...]
            m = jnp.min(v, axis=1, keepdims=True)
            jf = jnp.min(jnp.where(v == m, colf, jnp.float32(CAP)), axis=1)
            onehot = colf == jf[:, None]
            vals_ref[...] = jnp.where(onehot, jnp.float32(jnp.inf), v)
            idx_ref[0, pl.ds(t, 1), :] = (jf.astype(jnp.int32) + gbase)[None, :]
            return carry

        lax.fori_loop(0, KNN, step, 0, unroll=10)
        idx_ref[0, KNN:KPAD, :] = jnp.zeros((KPAD - KNN, RB), jnp.int32) + gbase

    @pl.when(r * RB >= slen)
    def _padding_rows():
        idx_ref[0] = jnp.zeros((KPAD, RB), jnp.int32) + gbase


def _knn(xp, maskcol, seg_len):
    din = xp.shape[-1]
    grid = (xp.shape[0], CAP // RB)
    return pl.pallas_call(
        _knn_body,
        grid=grid,
        in_specs=[
            pl.BlockSpec(memory_space=pltpu.SMEM),
            pl.BlockSpec((1, RB, din), lambda b, r: (b, r, 0)),
            pl.BlockSpec((1, CAP, din), lambda b, r: (b, 0, 0)),
            pl.BlockSpec((1, 1, CAP), lambda b, r: (b, 0, 0)),
        ],
        out_specs=pl.BlockSpec((1, KPAD, RB), lambda b, r: (b, 0, r)),
        out_shape=jax.ShapeDtypeStruct((xp.shape[0], KPAD, CAP), jnp.int32),
        scratch_shapes=[pltpu.VMEM((RB, CAP), jnp.float32)],
    )(seg_len, xp, xp, maskcol)


def _gather_rows(table, idx):
    n_rows = idx.shape[0]
    d = table.shape[1]
    info = plsc.get_sparse_core_info()
    nw = info.num_cores * info.num_subcores
    per_w = n_rows // nw
    ch = 128
    n_ch = per_w // ch
    mesh = plsc.VectorSubcoreMesh(core_axis_name="c", subcore_axis_name="s")

    nbuf = 8
    ngroups = n_ch // nbuf

    @functools.partial(
        pl.kernel, mesh=mesh,
        compiler_params=pltpu.CompilerParams(use_tc_tiling_on_sc=False),
        out_type=jax.ShapeDtypeStruct((n_rows, d), jnp.float32),
        scratch_types=(
            [pltpu.VMEM((per_w,), jnp.int32)]
            + [pltpu.VMEM((ch, d), jnp.float32) for _ in range(nbuf)]
            + [pltpu.SemaphoreType.DMA for _ in range(2 * nbuf)]
        ),
    )
    def k(table_hbm, idx_hbm, out_hbm, *bufs):
        idx_all = bufs[0]
        rows_v = bufs[1:1 + nbuf]
        gsem = bufs[1 + nbuf:1 + 2 * nbuf]
        ssem = bufs[1 + 2 * nbuf:]
        wid = lax.axis_index("s") * info.num_cores + lax.axis_index("c")
        base = wid * per_w
        pltpu.sync_copy(idx_hbm.at[pl.ds(base, per_w)], idx_all)

        def gath(c, u):
            pltpu.async_copy(table_hbm.at[idx_all.at[pl.ds(c * ch, ch)]],
                             rows_v[u], gsem[u])

        def gwait(c, u):
            pltpu.make_async_copy(table_hbm.at[idx_all.at[pl.ds(c * ch, ch)]],
                                  rows_v[u], gsem[u]).wait()

        def scat(c, u):
            pltpu.async_copy(rows_v[u], out_hbm.at[pl.ds(base + c * ch, ch)],
                             ssem[u])

        def swait(c, u):
            pltpu.make_async_copy(rows_v[u],
                                  out_hbm.at[pl.ds(base + c * ch, ch)],
                                  ssem[u]).wait()

        for u in range(nbuf):
            gath(u, u)

        def body(g, carry):
            for u in range(nbuf):
                gwait(g * nbuf + u, u)
                scat(g * nbuf + u, u)

            @pl.when(g < ngroups - 1)
            def _refill():
                for u in range(nbuf):
                    swait(g * nbuf + u, u)
                    gath((g + 1) * nbuf + u, u)

            return carry

        lax.fori_loop(0, ngroups, body, 0)
        for u in range(nbuf):
            swait((ngroups - 1) * nbuf + u, u)

    return k(table, idx)


def _edge_body(xi_ref, g_ref, w0a_ref, w0b_ref, b0_ref, w1_ref, b1_ref,
               o_ref):
    din = xi_ref.shape[-1]
    xi = xi_ref[0]
    g = g_ref[0]
    e = (g - xi[None, :, :]).reshape(KPAD * RB, din)
    h1 = _mm(xi, w0a_ref[...])[None, :, :] + (
        _mm(e, w0b_ref[...]).reshape(KPAD, RB, 64))
    h1 = jnp.maximum(h1 + b0_ref[...], 0.0).reshape(KPAD * RB, 64)
    h2 = jnp.maximum(_mm(h1, w1_ref[...]) + b1_ref[...], 0.0)
    o_ref[0] = jnp.max(h2.reshape(KPAD, RB, 64)[:KNN], axis=0)


def _edge_mlp_max(xp, g4, w0a, w0b, b0, w1, b1):
    din = xp.shape[-1]
    grid = (xp.shape[0], CAP // RB)
    return pl.pallas_call(
        _edge_body,
        grid=grid,
        in_specs=[
            pl.BlockSpec((1, RB, din), lambda b, r: (b, r, 0)),
            pl.BlockSpec((1, KPAD, RB, din), lambda b, r: (b, 0, r, 0)),
            pl.BlockSpec((din, 64), lambda b, r: (0, 0)),
            pl.BlockSpec((din, 64), lambda b, r: (0, 0)),
            pl.BlockSpec((1, 64), lambda b, r: (0, 0)),
            pl.BlockSpec((64, 64), lambda b, r: (0, 0)),
            pl.BlockSpec((1, 64), lambda b, r: (0, 0)),
        ],
        out_specs=pl.BlockSpec((1, RB, 64), lambda b, r: (b, r, 0)),
        out_shape=jax.ShapeDtypeStruct((xp.shape[0], CAP, 64), jnp.float32),
    )(xp, g4, w0a, w0b, b0, w1, b1)


def _head_body(x1_ref, x2_ref, x3_ref, w0_ref, b0_ref, w1_ref, b1_ref,
               w2_ref, b2_ref, wf_ref, bf_ref, o_ref):
    w0 = w0_ref[...]
    h = (_mm(x1_ref[...], w0[0:64]) + _mm(x2_ref[...], w0[64:128])
         + _mm(x3_ref[...], w0[128:192]) + b0_ref[...])
    h = jnp.maximum(h, 0.0)
    h = jnp.maximum(_mm(h, w1_ref[...]) + b1_ref[...], 0.0)
    h = jnp.maximum(_mm(h, w2_ref[...]) + b2_ref[...], 0.0)
    o_ref[...] = _mm(h, wf_ref[...]) + bf_ref[...]


def _head(x1f, x2f, x3f, w0, b0, w1, b1, w2, b2, wf, bf):
    grid = (NROW // RB,)
    c = lambda i: (0, 0)
    return pl.pallas_call(
        _head_body,
        grid=grid,
        in_specs=[
            pl.BlockSpec((RB, 64), lambda i: (i, 0)),
            pl.BlockSpec((RB, 64), lambda i: (i, 0)),
            pl.BlockSpec((RB, 64), lambda i: (i, 0)),
            pl.BlockSpec((192, 1024), c),
            pl.BlockSpec((1, 1024), c),
            pl.BlockSpec((1024, 256), c),
            pl.BlockSpec((1, 256), c),
            pl.BlockSpec((256, 128), c),
            pl.BlockSpec((1, 128), c),
            pl.BlockSpec((128, 3), c),
            pl.BlockSpec((1, 3), c),
        ],
        out_specs=pl.BlockSpec((RB, 3), lambda i: (i, 0)),
        out_shape=jax.ShapeDtypeStruct((NROW, 3), jnp.float32),
    )(x1f, x2f, x3f, w0, b0, w1, b1, w2, b2, wf, bf)


def _edge_conv(xp, maskcol, seg_len, w0, b0, w1, b1):
    din_real = w0.shape[0] // 2
    din = xp.shape[-1]
    w0a, w0b = w0[:din_real], w0[din_real:]
    if din_real < din:
        padr = din - din_real
        w0a = jnp.pad(w0a, ((0, padr), (0, 0)))
        w0b = jnp.pad(w0b, ((0, padr), (0, 0)))
    nseg = xp.shape[0]
    idx = _knn(xp, maskcol, seg_len)
    g = _gather_rows(xp.reshape(nseg * CAP, din),
                     idx.reshape(nseg * KPAD * CAP))
    g4 = g.reshape(nseg, KPAD, CAP, din)
    return _edge_mlp_max(xp, g4, w0a, w0b, b0.reshape(1, 64), w1,
                         b1.reshape(1, 64))


def kernel(pos, batch, c1_w0, c1_b0, c1_w1, c1_b1, c2_w0, c2_b0, c2_w1,
           c2_b1, c3_w0, c3_b0, c3_w1, c3_b1, mlp_w0, mlp_b0, mlp_w1,
           mlp_b1, mlp_w2, mlp_b2, fin_w, fin_b):
    n = pos.shape[0]
    batch = batch.astype(jnp.int32)
    seg_ids = jnp.arange(SEG, dtype=jnp.int32)
    seg_start = jnp.searchsorted(batch, seg_ids, side="left").astype(jnp.int32)
    seg_len = (jnp.searchsorted(batch, seg_ids, side="right").astype(jnp.int32)
               - seg_start)
    lidx = jnp.arange(CAP, dtype=jnp.int32)[None, :]
    valid = lidx < seg_len[:, None]
    g2l = jnp.where(valid, seg_start[:, None] + lidx, 0)
    maskcol = jnp.where(valid, 0.0, jnp.inf).astype(jnp.float32)
    maskcol = maskcol.reshape(SEG, 1, CAP)

    posp = jnp.pad(pos, ((0, CAP), (0, 5)))
    xp = jnp.stack([lax.dynamic_slice(posp, (seg_start[b], 0), (CAP, 8))
                    for b in range(SEG)])
    xp = jnp.where(valid[..., None], xp, 0.0)

    x1 = _edge_conv(xp, maskcol, seg_len, c1_w0, c1_b0, c1_w1, c1_b1)
    x2 = _edge_conv(x1, maskcol, seg_len, c2_w0, c2_b0, c2_w1, c2_b1)
    x3 = _edge_conv(x2, maskcol, seg_len, c3_w0, c3_b0, c3_w1, c3_b1)

    out = _head(x1.reshape(NROW, 64), x2.reshape(NROW, 64),
                x3.reshape(NROW, 64), mlp_w0, mlp_b0.reshape(1, 1024),
                mlp_w1, mlp_b1.reshape(1, 256), mlp_w2,
                mlp_b2.reshape(1, 128), fin_w, fin_b.reshape(1, 3))

    rows = jnp.arange(n, dtype=jnp.int32)
    out_idx = batch * CAP + (rows - seg_start[batch])
    return jnp.take(out, out_idx, axis=0)


# speedup vs baseline: 1.1282x; 1.0054x over previous
import functools

import jax
import jax.numpy as jnp
from jax import lax
from jax.experimental import pallas as pl
from jax.experimental.pallas import tpu as pltpu
from jax.experimental.pallas import tpu_sc as plsc

SEG = 8
CAP = 1536
NROW = SEG * CAP
KNN = 30
KPAD = 32
RB = 256


def _b16(x):
    return x.astype(jnp.bfloat16)


def _mm(a, b):
    return jnp.dot(_b16(a), _b16(b), preferred_element_type=jnp.float32)


def _knn_body(slen_ref, xr_ref, xw_ref, m_ref, idx_ref, vals_ref):
    b = pl.program_id(0)
    r = pl.program_id(1)
    gbase = b * CAP
    slen = slen_ref[b]

    @pl.when(r * RB < slen)
    def _active():
        xr = xr_ref[0]
        xw = xw_ref[0]
        dot = lax.dot_general(_b16(xr), _b16(xw), (((1,), (1,)), ((), ())),
                              preferred_element_type=jnp.float32)
        rr = jnp.sum(xr * xr, axis=1, keepdims=True)
        ww = jnp.sum(xw * xw, axis=1)[None, :]
        vals_ref[...] = rr + ww - 2.0 * dot + m_ref[0]

        colf = lax.broadcasted_iota(jnp.int32, (RB, CAP), 1).astype(jnp.float32)

        def step(t, carry):
            v = vals_ref[...]
            m = jnp.min(v, axis=1, keepdims=True)
            jf = jnp.min(jnp.where(v == m, colf, jnp.float32(CAP)), axis=1)
            onehot = colf == jf[:, None]
            vals_ref[...] = jnp.where(onehot, jnp.float32(jnp.inf), v)
            idx_ref[0, pl.ds(t, 1), :] = (jf.astype(jnp.int32) + gbase)[None, :]
            return carry

        lax.fori_loop(0, KNN, step, 0, unroll=15)
        idx_ref[0, KNN:KPAD, :] = jnp.zeros((KPAD - KNN, RB), jnp.int32) + gbase

    @pl.when(r * RB >= slen)
    def _padding_rows():
        idx_ref[0] = jnp.zeros((KPAD, RB), jnp.int32) + gbase


def _knn(xp, maskcol, seg_len):
    din = xp.shape[-1]
    grid = (xp.shape[0], CAP // RB)
    return pl.pallas_call(
        _knn_body,
        grid=grid,
        in_specs=[
            pl.BlockSpec(memory_space=pltpu.SMEM),
            pl.BlockSpec((1, RB, din), lambda b, r: (b, r, 0)),
            pl.BlockSpec((1, CAP, din), lambda b, r: (b, 0, 0)),
            pl.BlockSpec((1, 1, CAP), lambda b, r: (b, 0, 0)),
        ],
        out_specs=pl.BlockSpec((1, KPAD, RB), lambda b, r: (b, 0, r)),
        out_shape=jax.ShapeDtypeStruct((xp.shape[0], KPAD, CAP), jnp.int32),
        scratch_shapes=[pltpu.VMEM((RB, CAP), jnp.float32)],
    )(seg_len, xp, xp, maskcol)


def _gather_rows(table, idx):
    n_rows = idx.shape[0]
    d = table.shape[1]
    info = plsc.get_sparse_core_info()
    nw = info.num_cores * info.num_subcores
    per_w = n_rows // nw
    ch = 128
    n_ch = per_w // ch
    mesh = plsc.VectorSubcoreMesh(core_axis_name="c", subcore_axis_name="s")

    nbuf = 8
    ngroups = n_ch // nbuf

    @functools.partial(
        pl.kernel, mesh=mesh,
        compiler_params=pltpu.CompilerParams(use_tc_tiling_on_sc=False),
        out_type=jax.ShapeDtypeStruct((n_rows, d), jnp.float32),
        scratch_types=(
            [pltpu.VMEM((per_w,), jnp.int32)]
            + [pltpu.VMEM((ch, d), jnp.float32) for _ in range(nbuf)]
            + [pltpu.SemaphoreType.DMA for _ in range(2 * nbuf)]
        ),
    )
    def k(table_hbm, idx_hbm, out_hbm, *bufs):
        idx_all = bufs[0]
        rows_v = bufs[1:1 + nbuf]
        gsem = bufs[1 + nbuf:1 + 2 * nbuf]
        ssem = bufs[1 + 2 * nbuf:]
        wid = lax.axis_index("s") * info.num_cores + lax.axis_index("c")
        base = wid * per_w
        pltpu.sync_copy(idx_hbm.at[pl.ds(base, per_w)], idx_all)

        def gath(c, u):
            pltpu.async_copy(table_hbm.at[idx_all.at[pl.ds(c * ch, ch)]],
                             rows_v[u], gsem[u])

        def gwait(c, u):
            pltpu.make_async_copy(table_hbm.at[idx_all.at[pl.ds(c * ch, ch)]],
                                  rows_v[u], gsem[u]).wait()

        def scat(c, u):
            pltpu.async_copy(rows_v[u], out_hbm.at[pl.ds(base + c * ch, ch)],
                             ssem[u])

        def swait(c, u):
            pltpu.make_async_copy(rows_v[u],
                                  out_hbm.at[pl.ds(base + c * ch, ch)],
                                  ssem[u]).wait()

        for u in range(nbuf):
            gath(u, u)

        def body(g, carry):
            for u in range(nbuf):
                gwait(g * nbuf + u, u)
                scat(g * nbuf + u, u)

            @pl.when(g < ngroups - 1)
            def _refill():
                for u in range(nbuf):
                    swait(g * nbuf + u, u)
                    gath((g + 1) * nbuf + u, u)

            return carry

        lax.fori_loop(0, ngroups, body, 0)
        for u in range(nbuf):
            swait((ngroups - 1) * nbuf + u, u)

    return k(table, idx)


def _edge_body(xi_ref, g_ref, w0a_ref, w0b_ref, b0_ref, w1_ref, b1_ref,
               o_ref):
    din = xi_ref.shape[-1]
    xi = xi_ref[0]
    g = g_ref[0]
    e = (g - xi[None, :, :]).reshape(KPAD * RB, din)
    h1 = _mm(xi, w0a_ref[...])[None, :, :] + (
        _mm(e, w0b_ref[...]).reshape(KPAD, RB, 64))
    h1 = jnp.maximum(h1 + b0_ref[...], 0.0).reshape(KPAD * RB, 64)
    h2 = jnp.maximum(_mm(h1, w1_ref[...]) + b1_ref[...], 0.0)
    o_ref[0] = jnp.max(h2.reshape(KPAD, RB, 64)[:KNN], axis=0)


def _edge_mlp_max(xp, g4, w0a, w0b, b0, w1, b1):
    din = xp.shape[-1]
    grid = (xp.shape[0], CAP // RB)
    return pl.pallas_call(
        _edge_body,
        grid=grid,
        in_specs=[
            pl.BlockSpec((1, RB, din), lambda b, r: (b, r, 0)),
            pl.BlockSpec((1, KPAD, RB, din), lambda b, r: (b, 0, r, 0)),
            pl.BlockSpec((din, 64), lambda b, r: (0, 0)),
            pl.BlockSpec((din, 64), lambda b, r: (0, 0)),
            pl.BlockSpec((1, 64), lambda b, r: (0, 0)),
            pl.BlockSpec((64, 64), lambda b, r: (0, 0)),
            pl.BlockSpec((1, 64), lambda b, r: (0, 0)),
        ],
        out_specs=pl.BlockSpec((1, RB, 64), lambda b, r: (b, r, 0)),
        out_shape=jax.ShapeDtypeStruct((xp.shape[0], CAP, 64), jnp.float32),
    )(xp, g4, w0a, w0b, b0, w1, b1)


def _head_body(x1_ref, x2_ref, x3_ref, w0_ref, b0_ref, w1_ref, b1_ref,
               w2_ref, b2_ref, wf_ref, bf_ref, o_ref):
    w0 = w0_ref[...]
    h = (_mm(x1_ref[...], w0[0:64]) + _mm(x2_ref[...], w0[64:128])
         + _mm(x3_ref[...], w0[128:192]) + b0_ref[...])
    h = jnp.maximum(h, 0.0)
    h = jnp.maximum(_mm(h, w1_ref[...]) + b1_ref[...], 0.0)
    h = jnp.maximum(_mm(h, w2_ref[...]) + b2_ref[...], 0.0)
    o_ref[...] = _mm(h, wf_ref[...]) + bf_ref[...]


def _head(x1f, x2f, x3f, w0, b0, w1, b1, w2, b2, wf, bf):
    grid = (NROW // RB,)
    c = lambda i: (0, 0)
    return pl.pallas_call(
        _head_body,
        grid=grid,
        in_specs=[
            pl.BlockSpec((RB, 64), lambda i: (i, 0)),
            pl.BlockSpec((RB, 64), lambda i: (i, 0)),
            pl.BlockSpec((RB, 64), lambda i: (i, 0)),
            pl.BlockSpec((192, 1024), c),
            pl.BlockSpec((1, 1024), c),
            pl.BlockSpec((1024, 256), c),
            pl.BlockSpec((1, 256), c),
            pl.BlockSpec((256, 128), c),
            pl.BlockSpec((1, 128), c),
            pl.BlockSpec((128, 3), c),
            pl.BlockSpec((1, 3), c),
        ],
        out_specs=pl.BlockSpec((RB, 3), lambda i: (i, 0)),
        out_shape=jax.ShapeDtypeStruct((NROW, 3), jnp.float32),
    )(x1f, x2f, x3f, w0, b0, w1, b1, w2, b2, wf, bf)


def _edge_conv(xp, maskcol, seg_len, w0, b0, w1, b1):
    din_real = w0.shape[0] // 2
    din = xp.shape[-1]
    w0a, w0b = w0[:din_real], w0[din_real:]
    if din_real < din:
        padr = din - din_real
        w0a = jnp.pad(w0a, ((0, padr), (0, 0)))
        w0b = jnp.pad(w0b, ((0, padr), (0, 0)))
    nseg = xp.shape[0]
    idx = _knn(xp, maskcol, seg_len)
    g = _gather_rows(xp.reshape(nseg * CAP, din),
                     idx.reshape(nseg * KPAD * CAP))
    g4 = g.reshape(nseg, KPAD, CAP, din)
    return _edge_mlp_max(xp, g4, w0a, w0b, b0.reshape(1, 64), w1,
                         b1.reshape(1, 64))


def kernel(pos, batch, c1_w0, c1_b0, c1_w1, c1_b1, c2_w0, c2_b0, c2_w1,
           c2_b1, c3_w0, c3_b0, c3_w1, c3_b1, mlp_w0, mlp_b0, mlp_w1,
           mlp_b1, mlp_w2, mlp_b2, fin_w, fin_b):
    n = pos.shape[0]
    batch = batch.astype(jnp.int32)
    seg_ids = jnp.arange(SEG, dtype=jnp.int32)
    seg_start = jnp.searchsorted(batch, seg_ids, side="left").astype(jnp.int32)
    seg_len = (jnp.searchsorted(batch, seg_ids, side="right").astype(jnp.int32)
               - seg_start)
    lidx = jnp.arange(CAP, dtype=jnp.int32)[None, :]
    valid = lidx < seg_len[:, None]
    g2l = jnp.where(valid, seg_start[:, None] + lidx, 0)
    maskcol = jnp.where(valid, 0.0, jnp.inf).astype(jnp.float32)
    maskcol = maskcol.reshape(SEG, 1, CAP)

    posp = jnp.pad(pos, ((0, CAP), (0, 5)))
    xp = jnp.stack([lax.dynamic_slice(posp, (seg_start[b], 0), (CAP, 8))
                    for b in range(SEG)])
    xp = jnp.where(valid[..., None], xp, 0.0)

    x1 = _edge_conv(xp, maskcol, seg_len, c1_w0, c1_b0, c1_w1, c1_b1)
    x2 = _edge_conv(x1, maskcol, seg_len, c2_w0, c2_b0, c2_w1, c2_b1)
    x3 = _edge_conv(x2, maskcol, seg_len, c3_w0, c3_b0, c3_w1, c3_b1)

    out = _head(x1.reshape(NROW, 64), x2.reshape(NROW, 64),
                x3.reshape(NROW, 64), mlp_w0, mlp_b0.reshape(1, 1024),
                mlp_w1, mlp_b1.reshape(1, 256), mlp_w2,
                mlp_b2.reshape(1, 128), fin_w, fin_b.reshape(1, 3))

    rows = jnp.arange(n, dtype=jnp.int32)
    out_idx = batch * CAP + (rows - seg_start[batch])
    return jnp.take(out, out_idx, axis=0)
